# Initial kernel scaffold; baseline (speedup 1.0000x reference)
import functools

import jax
import jax.numpy as jnp
from jax import lax
from jax.experimental import pallas as pl
from jax.experimental.pallas import tpu as pltpu
from jax.experimental.pallas import tpu_sc as plsc

KS = 5
NCB = 8
N = 10000
E = 320000
C = 32
K = KS ** 3

CH = 128
NCHUNK = E // CH
NWORK = 32
BASE_CHUNKS = NCHUNK // NWORK
EXTRA = NCHUNK - BASE_CHUNKS * NWORK
ROWS_PER_TILE = N // 16

_BITS = [[(s >> d) & 1 for d in range(3)] for s in range(NCB)]
_STRIDES = [1, KS, KS * KS]


_PREP_BL = 6400


def _prep_body(attr_ref, ei_ref, bas_ref, gidx_ref):
    a = attr_ref[...]
    v = a * float(KS - 1)
    bot = jnp.floor(v)
    frac = v - bot
    boti = jnp.clip(bot.astype(jnp.int32), 0, KS - 1)
    src = ei_ref[0:1, :]
    bas_rows, gid_rows = [], []
    for s in range(NCB):
        b = None
        wi = None
        for d in range(3):
            fd = frac[d:d + 1, :]
            factor = fd if _BITS[s][d] == 1 else 1.0 - fd
            b = factor if b is None else b * factor
            idx_d = jnp.clip(boti[d:d + 1, :] + _BITS[s][d], 0, KS - 1)
            term = idx_d * _STRIDES[d]
            wi = term if wi is None else wi + term
        bas_rows.append(b)
        gid_rows.append(src * K + wi)
    bas_ref[...] = jnp.concatenate(bas_rows, axis=0)
    gidx_ref[...] = jnp.concatenate(gid_rows, axis=0)


_prep_call = pl.pallas_call(
    _prep_body,
    grid=(E // _PREP_BL,),
    in_specs=[
        pl.BlockSpec((3, _PREP_BL), lambda i: (0, i)),
        pl.BlockSpec((2, _PREP_BL), lambda i: (0, i)),
    ],
    out_specs=[
        pl.BlockSpec((NCB, _PREP_BL), lambda i: (0, i)),
        pl.BlockSpec((NCB, _PREP_BL), lambda i: (0, i)),
    ],
    out_shape=[
        jax.ShapeDtypeStruct((NCB, E), jnp.float32),
        jax.ShapeDtypeStruct((NCB, E), jnp.int32),
    ],
)


_MM_BM = 400


def _mm_body(x_ref, wf_ref, r_ref, b_ref, t_ref, xr_ref):
    xb = x_ref[...]
    t_ref[...] = jnp.dot(xb, wf_ref[...],
                         preferred_element_type=jnp.float32,
                         precision=lax.Precision.HIGHEST)
    xr_ref[...] = jnp.dot(xb, r_ref[...],
                          preferred_element_type=jnp.float32,
                          precision=lax.Precision.HIGHEST) + b_ref[...]


_mm_call = pl.pallas_call(
    _mm_body,
    grid=(N // _MM_BM,),
    in_specs=[
        pl.BlockSpec((_MM_BM, C), lambda i: (i, 0)),
        pl.BlockSpec((C, K * C), lambda i: (0, 0)),
        pl.BlockSpec((C, C), lambda i: (0, 0)),
        pl.BlockSpec((1, C), lambda i: (0, 0)),
    ],
    out_specs=[
        pl.BlockSpec((_MM_BM, K * C), lambda i: (i, 0)),
        pl.BlockSpec((_MM_BM, C), lambda i: (i, 0)),
    ],
    out_shape=[
        jax.ShapeDtypeStruct((N, K * C), jnp.float32),
        jax.ShapeDtypeStruct((N, C), jnp.float32),
    ],
)


_mesh = plsc.VectorSubcoreMesh(core_axis_name="c", subcore_axis_name="s",
                               num_cores=2, num_subcores=16)


def _make_sc(with_count):
    def body(t_hbm, gidx_hbm, bas_hbm, ei_hbm, *rest):
        if with_count:
            (agg_out, cnt_out, idxv, basv, rows, msg, dstv, onesv,
             agg_sh, cnt_sh, gsem) = rest
        else:
            (agg_out, idxv, basv, rows, msg, dstv,
             agg_sh, gsem) = rest
            cnt_out = cnt_sh = onesv = None
        cid = lax.axis_index("c")
        sid = lax.axis_index("s")
        wid = cid * 16 + sid

        @pl.loop(0, CH)
        def _(j):
            msg[j, pl.ds(0, 16)] = jnp.zeros((16,), jnp.float32)
            msg[j, pl.ds(16, 16)] = jnp.zeros((16,), jnp.float32)

        row0 = sid * ROWS_PER_TILE
        for kb in range(4):
            pltpu.sync_copy(msg.at[pl.ds(0, CH)],
                            agg_sh.at[pl.ds(row0 + kb * CH, CH)])
        pltpu.sync_copy(msg.at[pl.ds(0, 113)],
                        agg_sh.at[pl.ds(row0 + 4 * CH, 113)])
        if with_count:
            for kb in range(4):
                pltpu.sync_copy(msg.at[pl.ds(0, CH), pl.ds(0, 16)],
                                cnt_sh.at[pl.ds(row0 + kb * CH, CH)])
            pltpu.sync_copy(msg.at[pl.ds(0, 113), pl.ds(0, 16)],
                            cnt_sh.at[pl.ds(row0 + 4 * CH, 113)])

            @pl.loop(0, CH)
            def _(j):
                onesv[j, pl.ds(0, 16)] = jnp.ones((16,), jnp.float32)

        plsc.subcore_barrier()

        nchunks = jnp.where(wid < EXTRA, BASE_CHUNKS + 1, BASE_CHUNKS)

        @pl.loop(0, nchunks)
        def _(i):
            cidx = wid + i * NWORK
            base = cidx * CH
            pltpu.sync_copy(gidx_hbm.at[:, pl.ds(base, CH)], idxv)
            pltpu.sync_copy(bas_hbm.at[:, pl.ds(base, CH)], basv)
            pltpu.sync_copy(ei_hbm.at[1, pl.ds(base, CH)], dstv)
            descs = [pltpu.async_copy(t_hbm.at[idxv.at[s]], rows.at[s], gsem)
                     for s in range(NCB)]
            for dsc in descs:
                dsc.wait()

            @pl.loop(0, CH)
            def _(j):
                acc0 = jnp.zeros((16,), jnp.float32)
                acc1 = jnp.zeros((16,), jnp.float32)
                for s in range(NCB):
                    b = basv[s, j]
                    acc0 = acc0 + rows[s, j, pl.ds(0, 16)] * b
                    acc1 = acc1 + rows[s, j, pl.ds(16, 16)] * b
                msg[j, pl.ds(0, 16)] = acc0
                msg[j, pl.ds(16, 16)] = acc1

            pltpu.sync_copy(msg, agg_sh.at[dstv], add=True)
            if with_count:
                pltpu.sync_copy(onesv, cnt_sh.at[dstv], add=True)

        plsc.subcore_barrier()
        pltpu.sync_copy(agg_sh.at[pl.ds(row0, ROWS_PER_TILE)],
                        agg_out.at[cid, pl.ds(row0, ROWS_PER_TILE)])
        if with_count:
            pltpu.sync_copy(cnt_sh.at[pl.ds(row0, ROWS_PER_TILE)],
                            cnt_out.at[cid, pl.ds(row0, ROWS_PER_TILE)])

    out_type = [jax.ShapeDtypeStruct((2, N, C), jnp.float32)]
    scratch = [
        pltpu.VMEM((NCB, CH), jnp.int32),
        pltpu.VMEM((NCB, CH), jnp.float32),
        pltpu.VMEM((NCB, CH, C), jnp.float32),
        pltpu.VMEM((CH, C), jnp.float32),
        pltpu.VMEM((CH,), jnp.int32),
    ]
    if with_count:
        out_type.append(jax.ShapeDtypeStruct((2, N, 16), jnp.float32))
        scratch.append(pltpu.VMEM((CH, 16), jnp.float32))
    scratch.append(pltpu.VMEM_SHARED((N, C), jnp.float32))
    if with_count:
        scratch.append(pltpu.VMEM_SHARED((N, 16), jnp.float32))
    scratch.append(pltpu.SemaphoreType.DMA)

    return pl.kernel(body, out_type=tuple(out_type), mesh=_mesh,
                     scratch_types=tuple(scratch))


_sc_first = _make_sc(True)
_sc_rest = _make_sc(False)


def _bn_tail(t, g_ref, be_ref, y_ref):
    m = jnp.mean(t, axis=0, keepdims=True)
    var = jnp.mean((t - m) ** 2, axis=0, keepdims=True)
    y_ref[...] = (t - m) / jnp.sqrt(var + 1e-5) * g_ref[...] + be_ref[...]


def _post1_body(p_ref, pc_ref, xr_ref, x_ref, g_ref, be_ref, y_ref, rc_ref):
    c16 = pc_ref[0] + pc_ref[1]
    r16 = 1.0 / jnp.maximum(c16, 1.0)
    rc = jnp.concatenate([r16, r16], axis=1)
    rc_ref[...] = rc
    agg = (p_ref[0] + p_ref[1]) * rc
    t = agg + xr_ref[...]
    t = jnp.where(t >= 0, t, 0.01 * t) + x_ref[...]
    _bn_tail(t, g_ref, be_ref, y_ref)


def _post2_body(p_ref, rc_ref, xr_ref, x_ref, g_ref, be_ref, y_ref):
    agg = (p_ref[0] + p_ref[1]) * rc_ref[...]
    t = agg + xr_ref[...]
    t = jnp.where(t >= 0, t, 0.01 * t) + x_ref[...]
    _bn_tail(t, g_ref, be_ref, y_ref)


def _post3_body(p_ref, rc_ref, xr_ref, x_ref, g_ref, be_ref, y_ref):
    agg = (p_ref[0] + p_ref[1]) * rc_ref[...]
    t = agg + xr_ref[...] + x_ref[...]
    _bn_tail(t, g_ref, be_ref, y_ref)


_post1_call = pl.pallas_call(
    _post1_body,
    out_shape=[
        jax.ShapeDtypeStruct((N, C), jnp.float32),
        jax.ShapeDtypeStruct((N, C), jnp.float32),
    ],
)

_post2_call = pl.pallas_call(
    _post2_body,
    out_shape=jax.ShapeDtypeStruct((N, C), jnp.float32),
)

_post3_call = pl.pallas_call(
    _post3_body,
    out_shape=jax.ShapeDtypeStruct((N, C), jnp.float32),
)


def kernel(patch_embs, edge_index, edge_attr,
           w1, r1, b1, g1, be1,
           w2, r2, b2, g2, be2,
           w3, r3, b3, g3, be3):
    ei = edge_index
    attr_t = edge_attr.T
    basis8, gidx8 = _prep_call(attr_t, ei)

    def layer(x, w, r, b, first):
        wf = w.transpose(1, 0, 2).reshape(C, K * C)
        t, xr = _mm_call(x, wf, r, b.reshape(1, C))
        t = t.reshape(N * K, C)
        if first:
            p, pc = _sc_first(t, gidx8, basis8, ei)
            return p, pc, xr
        p, = _sc_rest(t, gidx8, basis8, ei)
        return p, None, xr

    p, pc, xr = layer(patch_embs, w1, r1, b1, True)
    y1, rc = _post1_call(p, pc, xr, patch_embs,
                         g1.reshape(1, C), be1.reshape(1, C))
    p, _, xr = layer(y1, w2, r2, b2, False)
    y2 = _post2_call(p, rc, xr, y1, g2.reshape(1, C), be2.reshape(1, C))
    p, _, xr = layer(y2, w3, r3, b3, False)
    y3 = _post3_call(p, rc, xr, y2, g3.reshape(1, C), be3.reshape(1, C))
    return y3


# speedup vs baseline: 5.2925x; 5.2925x over previous
import functools

import jax
import jax.numpy as jnp
from jax import lax
from jax.experimental import pallas as pl
from jax.experimental.pallas import tpu as pltpu
from jax.experimental.pallas import tpu_sc as plsc

KS = 5
NCB = 8
N = 10000
E = 320000
C = 32
K = KS ** 3

CH = 128
NCHUNK = E // CH
NWORK = 32
BASE_CHUNKS = NCHUNK // NWORK
EXTRA = NCHUNK - BASE_CHUNKS * NWORK
ROWS_PER_TILE = 624

_BITS = [[(s >> d) & 1 for d in range(3)] for s in range(NCB)]
_STRIDES = [1, KS, KS * KS]


_PREP_BL = 6400


def _prep_body(attr_ref, ei_ref, bas_ref, gidx_ref):
    a = attr_ref[...]
    v = a * float(KS - 1)
    bot = jnp.floor(v)
    frac = v - bot
    boti = jnp.clip(bot.astype(jnp.int32), 0, KS - 1)
    src = ei_ref[0:1, :]
    bas_rows, gid_rows = [], []
    for s in range(NCB):
        b = None
        wi = None
        for d in range(3):
            fd = frac[d:d + 1, :]
            factor = fd if _BITS[s][d] == 1 else 1.0 - fd
            b = factor if b is None else b * factor
            idx_d = jnp.clip(boti[d:d + 1, :] + _BITS[s][d], 0, KS - 1)
            term = idx_d * _STRIDES[d]
            wi = term if wi is None else wi + term
        bas_rows.append(b)
        gid_rows.append(src * K + wi)
    bas_ref[...] = jnp.concatenate(bas_rows, axis=0)
    gidx_ref[...] = jnp.concatenate(gid_rows, axis=0)


_prep_call = pl.pallas_call(
    _prep_body,
    grid=(E // _PREP_BL,),
    in_specs=[
        pl.BlockSpec((3, _PREP_BL), lambda i: (0, i)),
        pl.BlockSpec((2, _PREP_BL), lambda i: (0, i)),
    ],
    out_specs=[
        pl.BlockSpec((NCB, _PREP_BL), lambda i: (0, i)),
        pl.BlockSpec((NCB, _PREP_BL), lambda i: (0, i)),
    ],
    out_shape=[
        jax.ShapeDtypeStruct((NCB, E), jnp.float32),
        jax.ShapeDtypeStruct((NCB, E), jnp.int32),
    ],
)


_MM_BM = 400


def _mm_body(x_ref, wf_ref, r_ref, b_ref, t_ref, xr_ref):
    xb = x_ref[...]
    t_ref[...] = jnp.dot(xb, wf_ref[...],
                         preferred_element_type=jnp.float32,
                         precision=lax.Precision.HIGHEST)
    xr_ref[...] = jnp.dot(xb, r_ref[...],
                          preferred_element_type=jnp.float32,
                          precision=lax.Precision.HIGHEST) + b_ref[...]


_mm_call = pl.pallas_call(
    _mm_body,
    grid=(N // _MM_BM,),
    in_specs=[
        pl.BlockSpec((_MM_BM, C), lambda i: (i, 0)),
        pl.BlockSpec((C, K * C), lambda i: (0, 0)),
        pl.BlockSpec((C, C), lambda i: (0, 0)),
        pl.BlockSpec((1, C), lambda i: (0, 0)),
    ],
    out_specs=[
        pl.BlockSpec((_MM_BM, K * C), lambda i: (i, 0)),
        pl.BlockSpec((_MM_BM, C), lambda i: (i, 0)),
    ],
    out_shape=[
        jax.ShapeDtypeStruct((N, K * C), jnp.float32),
        jax.ShapeDtypeStruct((N, C), jnp.float32),
    ],
)


_mesh = plsc.VectorSubcoreMesh(core_axis_name="c", subcore_axis_name="s",
                               num_cores=2, num_subcores=16)


def _make_sc(with_count):
    def body(t_hbm, gidx_hbm, bas_hbm, ei_hbm, *rest):
        if with_count:
            (agg_out, cnt_out, idxv, basv, rows, msg, dstv, onesv,
             agg_sh, cnt_sh, gsem) = rest
        else:
            (agg_out, idxv, basv, rows, msg, dstv,
             agg_sh, gsem) = rest
            cnt_out = cnt_sh = onesv = None
        cid = lax.axis_index("c")
        sid = lax.axis_index("s")
        wid = cid * 16 + sid

        @pl.loop(0, CH)
        def _(j):
            msg[j, pl.ds(0, 16)] = jnp.zeros((16,), jnp.float32)
            msg[j, pl.ds(16, 16)] = jnp.zeros((16,), jnp.float32)

        row0 = sid * ROWS_PER_TILE
        for kb in range(4):
            pltpu.sync_copy(msg.at[pl.ds(0, CH)],
                            agg_sh.at[pl.ds(row0 + kb * CH, CH)])
        pltpu.sync_copy(msg.at[pl.ds(0, 112)],
                        agg_sh.at[pl.ds(row0 + 4 * CH, 112)])

        @pl.when(sid == 15)
        def _():
            pltpu.sync_copy(msg.at[pl.ds(0, 16)],
                            agg_sh.at[pl.ds(16 * ROWS_PER_TILE, 16)])
        if with_count:
            for kb in range(4):
                pltpu.sync_copy(msg.at[pl.ds(0, CH)],
                                cnt_sh.at[pl.ds(row0 + kb * CH, CH)])
            pltpu.sync_copy(msg.at[pl.ds(0, 112)],
                            cnt_sh.at[pl.ds(row0 + 4 * CH, 112)])

            @pl.when(sid == 15)
            def _():
                pltpu.sync_copy(msg.at[pl.ds(0, 16)],
                                cnt_sh.at[pl.ds(16 * ROWS_PER_TILE, 16)])

            @pl.loop(0, CH)
            def _(j):
                onesv[j, pl.ds(0, 16)] = jnp.ones((16,), jnp.float32)
                onesv[j, pl.ds(16, 16)] = jnp.ones((16,), jnp.float32)

        plsc.subcore_barrier()

        nchunks = jnp.where(wid < EXTRA, BASE_CHUNKS + 1, BASE_CHUNKS)

        @pl.loop(0, nchunks)
        def _(i):
            cidx = wid + i * NWORK
            base = cidx * CH
            pltpu.sync_copy(gidx_hbm.at[:, pl.ds(base, CH)], idxv)
            pltpu.sync_copy(bas_hbm.at[:, pl.ds(base, CH)], basv)
            pltpu.sync_copy(ei_hbm.at[1, pl.ds(base, CH)], dstv)
            descs = [pltpu.async_copy(t_hbm.at[idxv.at[s]], rows.at[s], gsem)
                     for s in range(NCB)]
            for dsc in descs:
                dsc.wait()

            @pl.loop(0, CH)
            def _(j):
                acc0 = jnp.zeros((16,), jnp.float32)
                acc1 = jnp.zeros((16,), jnp.float32)
                jsplat = jnp.broadcast_to(j, (16,)).astype(jnp.int32)
                for s in range(NCB):
                    b = plsc.load_gather(basv.at[s], [jsplat])
                    acc0 = acc0 + rows[s, j, pl.ds(0, 16)] * b
                    acc1 = acc1 + rows[s, j, pl.ds(16, 16)] * b
                msg[j, pl.ds(0, 16)] = acc0
                msg[j, pl.ds(16, 16)] = acc1

            pltpu.sync_copy(msg, agg_sh.at[dstv], add=True)
            if with_count:
                pltpu.sync_copy(onesv, cnt_sh.at[dstv], add=True)

        plsc.subcore_barrier()
        pltpu.sync_copy(agg_sh.at[pl.ds(row0, ROWS_PER_TILE)],
                        agg_out.at[cid, pl.ds(row0, ROWS_PER_TILE)])

        @pl.when(sid == 15)
        def _():
            pltpu.sync_copy(agg_sh.at[pl.ds(16 * ROWS_PER_TILE, 16)],
                            agg_out.at[cid, pl.ds(16 * ROWS_PER_TILE, 16)])
        if with_count:
            pltpu.sync_copy(cnt_sh.at[pl.ds(row0, ROWS_PER_TILE)],
                            cnt_out.at[cid, pl.ds(row0, ROWS_PER_TILE)])

            @pl.when(sid == 15)
            def _():
                pltpu.sync_copy(cnt_sh.at[pl.ds(16 * ROWS_PER_TILE, 16)],
                                cnt_out.at[cid, pl.ds(16 * ROWS_PER_TILE, 16)])

    out_type = [jax.ShapeDtypeStruct((2, N, C), jnp.float32)]
    scratch = [
        pltpu.VMEM((NCB, CH), jnp.int32),
        pltpu.VMEM((NCB, CH), jnp.float32),
        pltpu.VMEM((NCB, CH, C), jnp.float32),
        pltpu.VMEM((CH, C), jnp.float32),
        pltpu.VMEM((CH,), jnp.int32),
    ]
    if with_count:
        out_type.append(jax.ShapeDtypeStruct((2, N, C), jnp.float32))
        scratch.append(pltpu.VMEM((CH, C), jnp.float32))
    scratch.append(pltpu.VMEM_SHARED((N, C), jnp.float32))
    if with_count:
        scratch.append(pltpu.VMEM_SHARED((N, C), jnp.float32))
    scratch.append(pltpu.SemaphoreType.DMA)

    return pl.kernel(body, out_type=tuple(out_type), mesh=_mesh,
                     scratch_types=tuple(scratch),
                     compiler_params=pltpu.CompilerParams(
                         use_tc_tiling_on_sc=False,
                         needs_layout_passes=False))


_sc_first = _make_sc(True)
_sc_rest = _make_sc(False)


def _bn_tail(t, g_ref, be_ref, y_ref):
    m = jnp.mean(t, axis=0, keepdims=True)
    var = jnp.mean((t - m) ** 2, axis=0, keepdims=True)
    y_ref[...] = (t - m) / jnp.sqrt(var + 1e-5) * g_ref[...] + be_ref[...]


def _post1_body(p_ref, pc_ref, xr_ref, x_ref, g_ref, be_ref, y_ref, rc_ref):
    rc = 1.0 / jnp.maximum(pc_ref[0] + pc_ref[1], 1.0)
    rc_ref[...] = rc
    agg = (p_ref[0] + p_ref[1]) * rc
    t = agg + xr_ref[...]
    t = jnp.where(t >= 0, t, 0.01 * t) + x_ref[...]
    _bn_tail(t, g_ref, be_ref, y_ref)


def _post2_body(p_ref, rc_ref, xr_ref, x_ref, g_ref, be_ref, y_ref):
    agg = (p_ref[0] + p_ref[1]) * rc_ref[...]
    t = agg + xr_ref[...]
    t = jnp.where(t >= 0, t, 0.01 * t) + x_ref[...]
    _bn_tail(t, g_ref, be_ref, y_ref)


def _post3_body(p_ref, rc_ref, xr_ref, x_ref, g_ref, be_ref, y_ref):
    agg = (p_ref[0] + p_ref[1]) * rc_ref[...]
    t = agg + xr_ref[...] + x_ref[...]
    _bn_tail(t, g_ref, be_ref, y_ref)


_post1_call = pl.pallas_call(
    _post1_body,
    out_shape=[
        jax.ShapeDtypeStruct((N, C), jnp.float32),
        jax.ShapeDtypeStruct((N, C), jnp.float32),
    ],
)

_post2_call = pl.pallas_call(
    _post2_body,
    out_shape=jax.ShapeDtypeStruct((N, C), jnp.float32),
)

_post3_call = pl.pallas_call(
    _post3_body,
    out_shape=jax.ShapeDtypeStruct((N, C), jnp.float32),
)


def kernel(patch_embs, edge_index, edge_attr,
           w1, r1, b1, g1, be1,
           w2, r2, b2, g2, be2,
           w3, r3, b3, g3, be3):
    ei = edge_index
    attr_t = edge_attr.T
    basis8, gidx8 = _prep_call(attr_t, ei)

    def layer(x, w, r, b, first):
        wf = w.transpose(1, 0, 2).reshape(C, K * C)
        t, xr = _mm_call(x, wf, r, b.reshape(1, C))
        t = t.reshape(N * K, C)
        if first:
            p, pc = _sc_first(t, gidx8, basis8, ei)
            return p, pc, xr
        p, = _sc_rest(t, gidx8, basis8, ei)
        return p, None, xr

    p, pc, xr = layer(patch_embs, w1, r1, b1, True)
    y1, rc = _post1_call(p, pc, xr, patch_embs,
                         g1.reshape(1, C), be1.reshape(1, C))
    p, _, xr = layer(y1, w2, r2, b2, False)
    y2 = _post2_call(p, rc, xr, y1, g2.reshape(1, C), be2.reshape(1, C))
    p, _, xr = layer(y2, w3, r3, b3, False)
    y3 = _post3_call(p, rc, xr, y2, g3.reshape(1, C), be3.reshape(1, C))
    return y3


# speedup vs baseline: 7.7992x; 1.4736x over previous
import functools

import jax
import jax.numpy as jnp
from jax import lax
from jax.experimental import pallas as pl
from jax.experimental.pallas import tpu as pltpu
from jax.experimental.pallas import tpu_sc as plsc

KS = 5
NCB = 8
N = 10000
E = 320000
C = 32
K = KS ** 3

CH = 128
NCHUNK = E // CH
NWORK = 32
BASE_CHUNKS = NCHUNK // NWORK
EXTRA = NCHUNK - BASE_CHUNKS * NWORK
ROWS_PER_TILE = 624

_BITS = [[(s >> d) & 1 for d in range(3)] for s in range(NCB)]
_STRIDES = [1, KS, KS * KS]


_PREP_BL = 6400


def _prep_body(attr_ref, ei_ref, bas_ref, gidx_ref):
    a = attr_ref[...]
    v = a * float(KS - 1)
    bot = jnp.floor(v)
    frac = v - bot
    boti = jnp.clip(bot.astype(jnp.int32), 0, KS - 1)
    src = ei_ref[0:1, :]
    bas_rows, gid_rows = [], []
    for s in range(NCB):
        b = None
        wi = None
        for d in range(3):
            fd = frac[d:d + 1, :]
            factor = fd if _BITS[s][d] == 1 else 1.0 - fd
            b = factor if b is None else b * factor
            idx_d = jnp.clip(boti[d:d + 1, :] + _BITS[s][d], 0, KS - 1)
            term = idx_d * _STRIDES[d]
            wi = term if wi is None else wi + term
        bas_rows.append(b)
        gid_rows.append(src * K + wi)
    bas_ref[...] = jnp.concatenate(bas_rows, axis=0)
    gidx_ref[...] = jnp.concatenate(gid_rows, axis=0)


_prep_call = pl.pallas_call(
    _prep_body,
    grid=(E // _PREP_BL,),
    in_specs=[
        pl.BlockSpec((3, _PREP_BL), lambda i: (0, i)),
        pl.BlockSpec((2, _PREP_BL), lambda i: (0, i)),
    ],
    out_specs=[
        pl.BlockSpec((NCB, _PREP_BL), lambda i: (0, i)),
        pl.BlockSpec((NCB, _PREP_BL), lambda i: (0, i)),
    ],
    out_shape=[
        jax.ShapeDtypeStruct((NCB, E), jnp.float32),
        jax.ShapeDtypeStruct((NCB, E), jnp.int32),
    ],
)


_MM_BM = 400


def _mm_body(x_ref, wf_ref, r_ref, b_ref, t_ref, xr_ref):
    xb = x_ref[...]
    t_ref[...] = jnp.dot(xb, wf_ref[...],
                         preferred_element_type=jnp.float32,
                         precision=lax.Precision.HIGHEST)
    xr_ref[...] = jnp.dot(xb, r_ref[...],
                          preferred_element_type=jnp.float32,
                          precision=lax.Precision.HIGHEST) + b_ref[...]


_mm_call = pl.pallas_call(
    _mm_body,
    grid=(N // _MM_BM,),
    in_specs=[
        pl.BlockSpec((_MM_BM, C), lambda i: (i, 0)),
        pl.BlockSpec((C, K * C), lambda i: (0, 0)),
        pl.BlockSpec((C, C), lambda i: (0, 0)),
        pl.BlockSpec((1, C), lambda i: (0, 0)),
    ],
    out_specs=[
        pl.BlockSpec((_MM_BM, K * C), lambda i: (i, 0)),
        pl.BlockSpec((_MM_BM, C), lambda i: (i, 0)),
    ],
    out_shape=[
        jax.ShapeDtypeStruct((N, K * C), jnp.float32),
        jax.ShapeDtypeStruct((N, C), jnp.float32),
    ],
)


_mesh = plsc.VectorSubcoreMesh(core_axis_name="c", subcore_axis_name="s",
                               num_cores=2, num_subcores=16)


def _lane_bcast(vec, jj):
    idx = jnp.full((16, 1), jj, jnp.int32)
    dn = lax.GatherDimensionNumbers(offset_dims=(), collapsed_slice_dims=(0,),
                                    start_index_map=(0,))
    return lax.gather(vec, idx, dn, (1,),
                      mode=lax.GatherScatterMode.PROMISE_IN_BOUNDS)


def _make_sc(with_count):
    def body(t_hbm, gidx_hbm, bas_hbm, ei_hbm, *rest):
        if with_count:
            (agg_out, cnt_out, idxv, basv, rows, msg, dstv, onesv,
             agg_sh, cnt_sh, msem0, msem1, rsem0, rsem1) = rest
        else:
            (agg_out, idxv, basv, rows, msg, dstv,
             agg_sh, msem0, msem1, rsem0, rsem1) = rest
            cnt_out = cnt_sh = onesv = None
        msem = (msem0, msem1)
        rsem = (rsem0, rsem1)
        cid = lax.axis_index("c")
        sid = lax.axis_index("s")
        wid = cid * 16 + sid

        @pl.loop(0, CH)
        def _(j):
            msg[j, pl.ds(0, 16)] = jnp.zeros((16,), jnp.float32)
            msg[j, pl.ds(16, 16)] = jnp.zeros((16,), jnp.float32)

        row0 = sid * ROWS_PER_TILE
        for kb in range(4):
            pltpu.sync_copy(msg.at[pl.ds(0, CH)],
                            agg_sh.at[pl.ds(row0 + kb * CH, CH)])
        pltpu.sync_copy(msg.at[pl.ds(0, 112)],
                        agg_sh.at[pl.ds(row0 + 4 * CH, 112)])

        @pl.when(sid == 15)
        def _():
            pltpu.sync_copy(msg.at[pl.ds(0, 16)],
                            agg_sh.at[pl.ds(16 * ROWS_PER_TILE, 16)])
        if with_count:
            for kb in range(4):
                pltpu.sync_copy(msg.at[pl.ds(0, CH)],
                                cnt_sh.at[pl.ds(row0 + kb * CH, CH)])
            pltpu.sync_copy(msg.at[pl.ds(0, 112)],
                            cnt_sh.at[pl.ds(row0 + 4 * CH, 112)])

            @pl.when(sid == 15)
            def _():
                pltpu.sync_copy(msg.at[pl.ds(0, 16)],
                                cnt_sh.at[pl.ds(16 * ROWS_PER_TILE, 16)])

            @pl.loop(0, CH)
            def _(j):
                onesv[j, pl.ds(0, 16)] = jnp.ones((16,), jnp.float32)
                onesv[j, pl.ds(16, 16)] = jnp.ones((16,), jnp.float32)

        plsc.subcore_barrier()

        def chunk_of(k):
            return wid + k * NWORK

        def meta_start(c, p):
            base = c * CH
            pltpu.async_copy(gidx_hbm.at[:, pl.ds(base, CH)], idxv.at[p],
                             msem[p])
            pltpu.async_copy(bas_hbm.at[:, pl.ds(base, CH)], basv.at[p],
                             msem[p])
            pltpu.async_copy(ei_hbm.at[1, pl.ds(base, CH)], dstv.at[p],
                             msem[p])

        def meta_drain(p):
            pltpu.make_async_copy(gidx_hbm.at[:, pl.ds(0, CH)], idxv.at[p],
                                  msem[p]).wait()
            pltpu.make_async_copy(bas_hbm.at[:, pl.ds(0, CH)], basv.at[p],
                                  msem[p]).wait()
            pltpu.make_async_copy(ei_hbm.at[1, pl.ds(0, CH)], dstv.at[p],
                                  msem[p]).wait()

        def fire_gathers(p):
            for s in range(NCB):
                pltpu.async_copy(t_hbm.at[idxv.at[p, s]], rows.at[p, s],
                                 rsem[p])

        def rows_drain(p):
            for s in range(NCB):
                pltpu.make_async_copy(t_hbm.at[idxv.at[p, s]], rows.at[p, s],
                                      rsem[p]).wait()

        def compute_scatter(p):
            @pl.loop(0, CH, step=16)
            def _(g16):
                bvs = [basv[p, s, pl.ds(g16, 16)] for s in range(NCB)]
                for jj in range(16):
                    j = g16 + jj
                    acc0 = jnp.zeros((16,), jnp.float32)
                    acc1 = jnp.zeros((16,), jnp.float32)
                    for s in range(NCB):
                        b = _lane_bcast(bvs[s], jj)
                        acc0 = acc0 + rows[p, s, j, pl.ds(0, 16)] * b
                        acc1 = acc1 + rows[p, s, j, pl.ds(16, 16)] * b
                    msg[j, pl.ds(0, 16)] = acc0
                    msg[j, pl.ds(16, 16)] = acc1

            pltpu.sync_copy(msg, agg_sh.at[dstv.at[p]], add=True)
            if with_count:
                pltpu.sync_copy(onesv, cnt_sh.at[dstv.at[p]], add=True)

        meta_start(chunk_of(0), 0)
        meta_drain(0)
        fire_gathers(0)
        meta_start(chunk_of(1), 1)

        @pl.loop(0, BASE_CHUNKS, step=2)
        def _(i):
            for b in range(2):
                k = i + b
                p, q = b, 1 - b
                meta_drain(q)
                fire_gathers(q)
                rows_drain(p)
                compute_scatter(p)
                cn = jnp.minimum(chunk_of(k + 2), NCHUNK - 1)
                meta_start(cn, p)

        meta_drain(1)
        rows_drain(0)

        @pl.when(wid < EXTRA)
        def _():
            meta_start(BASE_CHUNKS * NWORK + wid, 0)
            meta_drain(0)
            fire_gathers(0)
            rows_drain(0)
            compute_scatter(0)

        plsc.subcore_barrier()
        pltpu.sync_copy(agg_sh.at[pl.ds(row0, ROWS_PER_TILE)],
                        agg_out.at[cid, pl.ds(row0, ROWS_PER_TILE)])

        @pl.when(sid == 15)
        def _():
            pltpu.sync_copy(agg_sh.at[pl.ds(16 * ROWS_PER_TILE, 16)],
                            agg_out.at[cid, pl.ds(16 * ROWS_PER_TILE, 16)])
        if with_count:
            pltpu.sync_copy(cnt_sh.at[pl.ds(row0, ROWS_PER_TILE)],
                            cnt_out.at[cid, pl.ds(row0, ROWS_PER_TILE)])

            @pl.when(sid == 15)
            def _():
                pltpu.sync_copy(cnt_sh.at[pl.ds(16 * ROWS_PER_TILE, 16)],
                                cnt_out.at[cid, pl.ds(16 * ROWS_PER_TILE, 16)])

    out_type = [jax.ShapeDtypeStruct((2, N, C), jnp.float32)]
    scratch = [
        pltpu.VMEM((2, NCB, CH), jnp.int32),
        pltpu.VMEM((2, NCB, CH), jnp.float32),
        pltpu.VMEM((2, NCB, CH, C), jnp.float32),
        pltpu.VMEM((CH, C), jnp.float32),
        pltpu.VMEM((2, CH), jnp.int32),
    ]
    if with_count:
        out_type.append(jax.ShapeDtypeStruct((2, N, C), jnp.float32))
        scratch.append(pltpu.VMEM((CH, C), jnp.float32))
    scratch.append(pltpu.VMEM_SHARED((N, C), jnp.float32))
    if with_count:
        scratch.append(pltpu.VMEM_SHARED((N, C), jnp.float32))
    scratch.extend([pltpu.SemaphoreType.DMA] * 4)

    return pl.kernel(body, out_type=tuple(out_type), mesh=_mesh,
                     scratch_types=tuple(scratch),
                     compiler_params=pltpu.CompilerParams(
                         use_tc_tiling_on_sc=False,
                         needs_layout_passes=False))


_sc_first = _make_sc(True)
_sc_rest = _make_sc(False)


def _bn_tail(t, g_ref, be_ref, y_ref):
    m = jnp.mean(t, axis=0, keepdims=True)
    var = jnp.mean((t - m) ** 2, axis=0, keepdims=True)
    y_ref[...] = (t - m) / jnp.sqrt(var + 1e-5) * g_ref[...] + be_ref[...]


def _post1_body(p_ref, pc_ref, xr_ref, x_ref, g_ref, be_ref, y_ref, rc_ref):
    rc = 1.0 / jnp.maximum(pc_ref[0] + pc_ref[1], 1.0)
    rc_ref[...] = rc
    agg = (p_ref[0] + p_ref[1]) * rc
    t = agg + xr_ref[...]
    t = jnp.where(t >= 0, t, 0.01 * t) + x_ref[...]
    _bn_tail(t, g_ref, be_ref, y_ref)


def _post2_body(p_ref, rc_ref, xr_ref, x_ref, g_ref, be_ref, y_ref):
    agg = (p_ref[0] + p_ref[1]) * rc_ref[...]
    t = agg + xr_ref[...]
    t = jnp.where(t >= 0, t, 0.01 * t) + x_ref[...]
    _bn_tail(t, g_ref, be_ref, y_ref)


def _post3_body(p_ref, rc_ref, xr_ref, x_ref, g_ref, be_ref, y_ref):
    agg = (p_ref[0] + p_ref[1]) * rc_ref[...]
    t = agg + xr_ref[...] + x_ref[...]
    _bn_tail(t, g_ref, be_ref, y_ref)


_post1_call = pl.pallas_call(
    _post1_body,
    out_shape=[
        jax.ShapeDtypeStruct((N, C), jnp.float32),
        jax.ShapeDtypeStruct((N, C), jnp.float32),
    ],
)

_post2_call = pl.pallas_call(
    _post2_body,
    out_shape=jax.ShapeDtypeStruct((N, C), jnp.float32),
)

_post3_call = pl.pallas_call(
    _post3_body,
    out_shape=jax.ShapeDtypeStruct((N, C), jnp.float32),
)


def kernel(patch_embs, edge_index, edge_attr,
           w1, r1, b1, g1, be1,
           w2, r2, b2, g2, be2,
           w3, r3, b3, g3, be3):
    ei = edge_index
    attr_t = edge_attr.T
    basis8, gidx8 = _prep_call(attr_t, ei)

    def layer(x, w, r, b, first):
        wf = w.transpose(1, 0, 2).reshape(C, K * C)
        t, xr = _mm_call(x, wf, r, b.reshape(1, C))
        t = t.reshape(N * K, C)
        if first:
            p, pc = _sc_first(t, gidx8, basis8, ei)
            return p, pc, xr
        p, = _sc_rest(t, gidx8, basis8, ei)
        return p, None, xr

    p, pc, xr = layer(patch_embs, w1, r1, b1, True)
    y1, rc = _post1_call(p, pc, xr, patch_embs,
                         g1.reshape(1, C), be1.reshape(1, C))
    p, _, xr = layer(y1, w2, r2, b2, False)
    y2 = _post2_call(p, rc, xr, y1, g2.reshape(1, C), be2.reshape(1, C))
    p, _, xr = layer(y2, w3, r3, b3, False)
    y3 = _post3_call(p, rc, xr, y2, g3.reshape(1, C), be3.reshape(1, C))
    return y3
